# Initial kernel scaffold; baseline (speedup 1.0000x reference)
import jax
import jax.numpy as jnp
from jax.experimental import pallas as pl


def kernel(x, edge_index, weight, amplitude, phase, period):
    raise NotImplementedError("write your pallas kernel here")


# speedup vs baseline: 440.2965x; 440.2965x over previous
import jax
import jax.numpy as jnp
from jax import lax
from jax.experimental import pallas as pl
from jax.experimental.pallas import tpu as pltpu
from jax.experimental.pallas import tpu_sc as plsc

N = 100000
NPAD = 100352
TPN = NPAD // 16
SUB = TPN // 4
E = 3200000
EPAD = 3211264
PWE = EPAD // 32
G = 8
NCHUNK = PWE // (G * 128)

_f32 = jnp.float32
_i32 = jnp.int32

_CP = pltpu.CompilerParams(needs_layout_passes=False, use_tc_tiling_on_sc=False)


def _mesh():
    return plsc.VectorSubcoreMesh(
        core_axis_name="c", subcore_axis_name="s", num_cores=2, num_subcores=16
    )


def _prep_body(xw_hbm, w_hbm, f_out, wbuf, xbuf, zbuf):
    cid = lax.axis_index("c")
    tid = lax.axis_index("s")
    wid = tid * 2 + cid
    half = TPN // 2
    lanes = lax.iota(_i32, 16)

    pltpu.sync_copy(w_hbm, wbuf)
    wv = wbuf[...]
    w1 = [wv[4 + p] for p in range(4)]
    w2 = [wv[8 + p] for p in range(4)]

    @pl.loop(0, 2)
    def _sub(s):
        base = wid * half + s * SUB
        for r in range(8):
            pltpu.sync_copy(
                xw_hbm.at[pl.ds(r * NPAD + base, SUB)],
                xbuf.at[pl.ds(r * SUB, SUB)],
            )

        @pl.loop(0, SUB // 16)
        def _grp(i):
            off = i * 16
            fidx = (off + lanes) * 4
            for b in range(2):
                xv = [xbuf[pl.ds((4 * b + p) * SUB + off, 16)] for p in range(4)]
                z1 = xv[0] * w1[0] + xv[1] * w1[1] + xv[2] * w1[2] + xv[3] * w1[3]
                z2 = xv[0] * w2[0] + xv[1] * w2[1] + xv[2] * w2[2] + xv[3] * w2[3]
                plsc.store_scatter(zbuf, [fidx + b], z1)
                plsc.store_scatter(zbuf, [fidx + (2 + b)], z2)

        pltpu.sync_copy(zbuf, f_out.at[pl.ds(4 * base, 4 * SUB)])


def _prep(xwf, wflat):
    return pl.kernel(
        _prep_body,
        out_type=jax.ShapeDtypeStruct((4 * NPAD,), _f32),
        mesh=_mesh(),
        compiler_params=_CP,
        scratch_types=[
            pltpu.VMEM((16,), _f32),
            pltpu.VMEM((8 * SUB,), _f32),
            pltpu.VMEM((4 * SUB,), _f32),
        ],
        name="sine_gpvar_prep",
    )(xwf, wflat)


def _make_hop_body(C):
    def _hop_body(f_hbm, src_hbm, dst_hbm, zeros_hbm, g_out,
                  g_sp, sidx, didx, rows, gsem, ssem):
        cid = lax.axis_index("c")
        tid = lax.axis_index("s")
        wid = tid * 2 + cid
        n0 = tid * TPN

        pltpu.sync_copy(zeros_hbm, g_sp.at[pl.ds(n0, TPN)])
        plsc.subcore_barrier()

        e0 = wid * (PWE // 128)

        @pl.loop(0, NCHUNK)
        def _chunk(s):
            r0 = e0 + s * G
            pltpu.sync_copy(src_hbm.at[pl.ds(r0, G)], sidx)
            pltpu.sync_copy(dst_hbm.at[pl.ds(r0, G)], didx)
            gds = []
            for j in range(G):
                gds.append(
                    pltpu.async_copy(
                        f_hbm.at[sidx.at[j]],
                        rows.at[pl.ds(j * 128, 128)], gsem,
                    )
                )
            for d in gds:
                d.wait()
            sds = []
            for j in range(G):
                sds.append(
                    pltpu.async_copy(
                        rows.at[pl.ds(j * 128, 128)],
                        g_sp.at[didx.at[j]], ssem,
                        add=True,
                    )
                )
            for d in sds:
                d.wait()

        plsc.subcore_barrier()

        pltpu.sync_copy(g_sp.at[pl.ds(n0, TPN)], g_out.at[cid].at[pl.ds(n0, TPN)])

    return _hop_body


def _hop(C, fT, srcE, dstE, zerosT):
    return pl.kernel(
        _make_hop_body(C),
        out_type=jax.ShapeDtypeStruct((2, NPAD, C), _f32),
        mesh=_mesh(),
        compiler_params=_CP,
        scratch_types=[
            pltpu.VMEM_SHARED((NPAD, C), _f32),
            pltpu.VMEM((G, 128), _i32),
            pltpu.VMEM((G, 128), _i32),
            pltpu.VMEM((G * 128, C), _f32),
            pltpu.SemaphoreType.DMA,
            pltpu.SemaphoreType.DMA,
        ],
        name=f"sine_gpvar_hop{C}",
    )(fT, srcE, dstE, zerosT)


def _mid_body(a_ref, f2_ref):
    f2_ref[...] = a_ref[0:1] + a_ref[1:2]


def _mid(a):
    BL = 2048
    return pl.pallas_call(
        _mid_body,
        grid=((2 * NPAD) // BL,),
        in_specs=[pl.BlockSpec((2, BL), lambda i: (0, i))],
        out_specs=pl.BlockSpec((1, BL), lambda i: (0, i)),
        out_shape=jax.ShapeDtypeStruct((1, 2 * NPAD), _f32),
    )(a)


def _epi_body(xw_ref, w_ref, g1_ref, g2_ref, amp_ref, ph_ref, o_ref):
    w = w_ref[...]
    xw = xw_ref[...].reshape(2, 4, xw_ref.shape[1])
    z0 = (xw * w[0][None, :, None]).sum(axis=1)
    u = z0 + g1_ref[0:2] + g1_ref[2:4] + g2_ref[0:2] + g2_ref[2:4]
    sine = amp_ref[...] * jnp.sin(ph_ref[...])
    o_ref[...] = jnp.tanh(u) + sine


def _epilogue(xwp, weight, g1p, g2p, amp, ph):
    BL = 512
    return pl.pallas_call(
        _epi_body,
        grid=(NPAD // BL,),
        in_specs=[
            pl.BlockSpec((8, BL), lambda i: (0, i)),
            pl.BlockSpec((3, 4), lambda i: (0, 0)),
            pl.BlockSpec((4, BL), lambda i: (0, i)),
            pl.BlockSpec((4, BL), lambda i: (0, i)),
            pl.BlockSpec((1, BL), lambda i: (0, i)),
            pl.BlockSpec((1, BL), lambda i: (0, i)),
        ],
        out_specs=pl.BlockSpec((2, BL), lambda i: (0, i)),
        out_shape=jax.ShapeDtypeStruct((2, NPAD), _f32),
    )(xwp, weight, g1p, g2p, amp, ph)


def kernel(x, edge_index, weight, amplitude, phase, period):
    del period
    xw = x[:, 4:, :, 0].reshape(8, N)
    xwp = jnp.pad(xw, ((0, 0), (0, NPAD - N)))
    xwf = xwp.reshape(8 * NPAD)
    epad = jnp.full((EPAD - E,), N, _i32)
    srcE = jnp.concatenate([edge_index[0], epad]).reshape(EPAD // 128, 128)
    dstE = jnp.concatenate([edge_index[1], epad]).reshape(EPAD // 128, 128)
    zeros8 = jnp.zeros((TPN, 8), _f32)
    wflat = jnp.pad(weight.reshape(12), (0, 4))

    f1 = jnp.pad(_prep(xwf, wflat).reshape(NPAD, 4), ((0, 0), (0, 4)))
    g1 = _hop(8, f1, srcE, dstE, zeros8)
    f2 = _mid(g1[:, :, 2:4].reshape(2, 2 * NPAD)).reshape(NPAD, 2)
    f2 = jnp.pad(f2, ((0, 0), (0, 6)))
    g1p = jnp.transpose(g1[:, :, 0:2], (0, 2, 1)).reshape(4, NPAD)
    g2 = _hop(8, f2, srcE, dstE, zeros8)
    g2p = jnp.transpose(g2[:, :, 0:2], (0, 2, 1)).reshape(4, NPAD)
    amp = jnp.pad(amplitude[:, 0], (0, NPAD - N)).reshape(1, NPAD)
    ph = jnp.pad(phase[:, 0], (0, NPAD - N)).reshape(1, NPAD)
    y = _epilogue(xwp, weight, g1p, g2p, amp, ph)
    return y[:, :N][:, None, :, None]
